# baseline (speedup 1.0000x reference)
import jax
import jax.numpy as jnp
from jax import lax
from jax.experimental import pallas as pl
from jax.experimental.pallas import tpu as pltpu
from jax.experimental.pallas import tpu_sc as plsc

_NC = 2
_NS = 16
_NW = _NC * _NS
_NBUF = 4
_BT = 128
_TPAD = 137


def _build(B, S, V, D):
    n_bt = B // _BT
    n_units = S * n_bt
    per_w = n_units // _NW
    mesh = plsc.VectorSubcoreMesh(core_axis_name="c", subcore_axis_name="s")

    def body(x_hbm, tab_hbm, pos_hbm, out_hbm, idx_v, rows_v, trans_v,
             trans2_v, pos_v, *sems):
        sg = sems[:_NBUF]
        ss = sems[_NBUF:]
        wid = lax.axis_index("s") * _NC + lax.axis_index("c")
        base = wid * per_w

        pltpu.sync_copy(pos_hbm, pos_v)

        lane = lax.iota(jnp.int32, 16)
        idx_dt = [(c * 16 + lane) >> 3 for c in range(D // 16)]
        idx_d = [c * 16 + lane for c in range(D // 16)]

        def unit_sbt(i):
            uid = base + i
            s = uid // n_bt
            return s, uid % n_bt

        def load_idx(i, u):
            s, bt = unit_sbt(i)
            pltpu.sync_copy(x_hbm.at[s >> 3, bt, s & 7], idx_v.at[u])

        def start_gather(u):
            pltpu.async_copy(tab_hbm.at[idx_v.at[u]], rows_v.at[u], sg[u])

        def wait_gather(u):
            pltpu.make_async_copy(
                tab_hbm.at[idx_v.at[u]], rows_v.at[u], sg[u]).wait()

        def start_scatter(i, u):
            s, bt = unit_sbt(i)
            for dt in range(D // 8):
                pltpu.async_copy(
                    trans2_v.at[u, dt], out_hbm.at[s, dt, bt], ss[u])

        def wait_scatter(u):
            pltpu.make_async_copy(
                trans2_v.at[u], out_hbm.at[0, :, 0], ss[u]).wait()

        for u in range(2):
            load_idx(u, u)
            start_gather(u)

        @pl.loop(0, per_w, step=_NBUF)
        def _(i0):
            for u in range(_NBUF):
                i = i0 + u
                un = (u + 2) % _NBUF

                @pl.when(i + 2 < per_w)
                def _():
                    load_idx(i + 2, un)
                    start_gather(un)

                @pl.when(i >= _NBUF)
                def _():
                    wait_scatter(u)

                wait_gather(u)

                s, _bt = unit_sbt(i)
                pos_c = [pos_v[s, pl.ds(c * 16, 16)] for c in range(D // 16)]

                @pl.loop(0, _BT, unroll=2)
                def _(t):
                    tv = jnp.full((16,), t, jnp.int32)
                    for c in range(D // 16):
                        v = rows_v[u, t, pl.ds(c * 16, 16)] + pos_c[c]
                        plsc.store_scatter(
                            trans_v.at[u], [idx_d[c], tv], v)

                @pl.loop(0, D, unroll=2)
                def _(d):
                    for c2 in range(_BT // 16):
                        trans2_v[u, d >> 3, d & 7, pl.ds(c2 * 16, 16)] = (
                            trans_v[u, d, pl.ds(c2 * 16, 16)])

                start_scatter(i, u)

        for u in range(_NBUF):
            wait_scatter((per_w - _NBUF + u) % _NBUF)

    return pl.kernel(
        body,
        out_type=jax.ShapeDtypeStruct((S, D // 8, B // _BT, 8, _BT),
                                      jnp.float32),
        mesh=mesh,
        compiler_params=pltpu.CompilerParams(
            use_tc_tiling_on_sc=False, needs_layout_passes=False),
        scratch_types=[
            pltpu.VMEM((_NBUF, _BT), jnp.int32),
            pltpu.VMEM((_NBUF, _BT, D), jnp.float32),
            pltpu.VMEM((_NBUF, D, _TPAD), jnp.float32),
            pltpu.VMEM((_NBUF, D // 8, 8, _BT), jnp.float32),
            pltpu.VMEM((S, D), jnp.float32),
        ] + [pltpu.SemaphoreType.DMA] * (2 * _NBUF),
    )


def kernel(x, token_table, pos_table):
    B, S = x.shape
    V, D = token_table.shape
    x4 = (x.astype(jnp.int32).T
          .reshape(S // 8, 8, B // _BT, _BT)
          .transpose(0, 2, 1, 3))
    out5 = _build(B, S, V, D)(x4, token_table, pos_table)
    return out5.transpose(2, 4, 0, 1, 3).reshape(B, S, D)


# speedup vs baseline: 1.3867x; 1.3867x over previous
import jax
import jax.numpy as jnp
from jax import lax
from jax.experimental import pallas as pl
from jax.experimental.pallas import tpu as pltpu
from jax.experimental.pallas import tpu_sc as plsc

_NC = 2
_NS = 16
_NW = _NC * _NS
_NBUF = 4
_BT = 128
_TPAD = 137


def _build(B, S, V, D):
    n_bt = B // _BT
    n_units = S * n_bt
    per_w = n_units // _NW
    mesh = plsc.VectorSubcoreMesh(core_axis_name="c", subcore_axis_name="s")

    def body(x_hbm, tab_hbm, pos_hbm, out_hbm, idx_v, rows_v, trans_v,
             pos_v, *sems):
        sg = sems[:_NBUF]
        ss = sems[_NBUF:]
        wid = lax.axis_index("s") * _NC + lax.axis_index("c")
        base = wid * per_w

        pltpu.sync_copy(pos_hbm, pos_v)

        lane = lax.iota(jnp.int32, 16)
        idx_dt = [(c * 16 + lane) >> 3 for c in range(D // 16)]
        idx_dt = [(c * 16 + lane) >> 3 for c in range(D // 16)]
        idx_dr = [(c * 16 + lane) & 7 for c in range(D // 16)]

        def unit_sbt(i):
            uid = base + i
            s = uid // n_bt
            return s, uid % n_bt

        def load_idx(i, u):
            s, bt = unit_sbt(i)
            pltpu.sync_copy(x_hbm.at[s >> 3, bt, s & 7], idx_v.at[u])

        def start_gather(u):
            pltpu.async_copy(tab_hbm.at[idx_v.at[u]], rows_v.at[u], sg[u])

        def wait_gather(u):
            pltpu.make_async_copy(
                tab_hbm.at[idx_v.at[u]], rows_v.at[u], sg[u]).wait()

        def start_scatter(i, u):
            s, bt = unit_sbt(i)
            pltpu.async_copy(trans_v.at[u, :, :, pl.ds(0, _BT)],
                             out_hbm.at[s, :, bt], ss[u])

        def wait_scatter(u):
            pltpu.make_async_copy(trans_v.at[u, :, :, pl.ds(0, _BT)],
                                  out_hbm.at[0, :, 0], ss[u]).wait()

        for u in range(2):
            load_idx(u, u)
            start_gather(u)

        @pl.loop(0, per_w, step=_NBUF)
        def _(i0):
            for u in range(_NBUF):
                i = i0 + u
                un = (u + 2) % _NBUF

                @pl.when(i + 2 < per_w)
                def _():
                    load_idx(i + 2, un)
                    start_gather(un)

                @pl.when(i >= _NBUF)
                def _():
                    wait_scatter(u)

                wait_gather(u)

                s, _bt = unit_sbt(i)
                pos_c = [pos_v[s, pl.ds(c * 16, 16)] for c in range(D // 16)]

                @pl.loop(0, _BT, unroll=8)
                def _(t):
                    tv = jnp.full((16,), t, jnp.int32)
                    for c in range(D // 16):
                        v = rows_v[u, t, pl.ds(c * 16, 16)] + pos_c[c]
                        plsc.store_scatter(
                            trans_v.at[u], [idx_dt[c], idx_dr[c], tv], v)

                start_scatter(i, u)

        for u in range(_NBUF):
            wait_scatter((per_w - _NBUF + u) % _NBUF)

    return pl.kernel(
        body,
        out_type=jax.ShapeDtypeStruct((S, D // 8, B // _BT, 8, _BT),
                                      jnp.float32),
        mesh=mesh,
        compiler_params=pltpu.CompilerParams(
            use_tc_tiling_on_sc=False, needs_layout_passes=False),
        scratch_types=[
            pltpu.VMEM((_NBUF, _BT), jnp.int32),
            pltpu.VMEM((_NBUF, _BT, D), jnp.float32),
            pltpu.VMEM((_NBUF, D // 8, 8, _TPAD), jnp.float32),
            pltpu.VMEM((S, D), jnp.float32),
        ] + [pltpu.SemaphoreType.DMA] * (2 * _NBUF),
    )


def kernel(x, token_table, pos_table):
    B, S = x.shape
    V, D = token_table.shape
    x4 = (x.astype(jnp.int32).T
          .reshape(S // 8, 8, B // _BT, _BT)
          .transpose(0, 2, 1, 3))
    out5 = _build(B, S, V, D)(x4, token_table, pos_table)
    return out5.transpose(2, 4, 0, 1, 3).reshape(B, S, D)


# speedup vs baseline: 1.8002x; 1.2982x over previous
import jax
import jax.numpy as jnp
from jax import lax
from jax.experimental import pallas as pl
from jax.experimental.pallas import tpu as pltpu
from jax.experimental.pallas import tpu_sc as plsc

_NC = 2
_NS = 16
_NW = _NC * _NS
_NBUF = 4
_BT = 128
_TPAD = 136


def _build(B, S, V, D):
    n_bt = B // _BT
    n_units = S * n_bt
    per_w = n_units // _NW
    mesh = plsc.VectorSubcoreMesh(core_axis_name="c", subcore_axis_name="s")

    def body(x_hbm, tab_hbm, pos_hbm, out_hbm, idx_v, rows_v, trans_v,
             pos_v, *sems):
        sg = sems[:_NBUF]
        ss = sems[_NBUF:2 * _NBUF]
        si = sems[2 * _NBUF:]
        wid = lax.axis_index("s") * _NC + lax.axis_index("c")
        base = wid * per_w

        pltpu.sync_copy(pos_hbm, pos_v)

        lane = lax.iota(jnp.int32, 16)
        idx_dt = [(c * 16 + lane) >> 3 for c in range(D // 16)]
        idx_dt = [(c * 16 + lane) >> 3 for c in range(D // 16)]
        idx_dr = [(c * 16 + lane) & 7 for c in range(D // 16)]

        def unit_sbt(i):
            uid = base + i
            s = uid // n_bt
            return s, uid % n_bt

        def load_idx(i, u):
            s, bt = unit_sbt(i)
            pltpu.async_copy(x_hbm.at[s >> 3, bt, s & 7], idx_v.at[u], si[u])

        def wait_idx(u):
            pltpu.make_async_copy(
                x_hbm.at[0, 0, 0], idx_v.at[u], si[u]).wait()

        def start_gather(u):
            pltpu.async_copy(tab_hbm.at[idx_v.at[u]], rows_v.at[u], sg[u])

        def wait_gather(u):
            pltpu.make_async_copy(
                tab_hbm.at[idx_v.at[u]], rows_v.at[u], sg[u]).wait()

        def start_scatter(i, u):
            s, bt = unit_sbt(i)
            pltpu.async_copy(trans_v.at[u, :, :, pl.ds(0, _BT)],
                             out_hbm.at[s, :, bt], ss[u])

        def wait_scatter(u):
            pltpu.make_async_copy(trans_v.at[u, :, :, pl.ds(0, _BT)],
                                  out_hbm.at[0, :, 0], ss[u]).wait()

        for u in range(_NBUF):
            load_idx(u, u)
        for u in range(2):
            wait_idx(u)
            start_gather(u)

        @pl.loop(0, per_w, step=_NBUF)
        def _(i0):
            for u in range(_NBUF):
                i = i0 + u
                un = (u + 2) % _NBUF

                @pl.when(i + 2 < per_w)
                def _():
                    wait_idx(un)
                    start_gather(un)

                @pl.when(i >= _NBUF)
                def _():
                    wait_scatter(u)

                wait_gather(u)

                @pl.when(i + _NBUF < per_w)
                def _():
                    load_idx(i + _NBUF, u)

                s, _bt = unit_sbt(i)
                pos_c = [pos_v[s, pl.ds(c * 16, 16)] for c in range(D // 16)]

                @pl.loop(0, _BT)
                def _(t):
                    tv = jnp.full((16,), t, jnp.int32)
                    for c in range(D // 16):
                        v = rows_v[u, t, pl.ds(c * 16, 16)] + pos_c[c]
                        plsc.store_scatter(
                            trans_v.at[u], [idx_dt[c], idx_dr[c], tv], v)

                start_scatter(i, u)

        for u in range(_NBUF):
            wait_scatter((per_w - _NBUF + u) % _NBUF)

    return pl.kernel(
        body,
        out_type=jax.ShapeDtypeStruct((S, D // 8, B // _BT, 8, _BT),
                                      jnp.float32),
        mesh=mesh,
        compiler_params=pltpu.CompilerParams(
            use_tc_tiling_on_sc=False, needs_layout_passes=False),
        scratch_types=[
            pltpu.VMEM((_NBUF, _BT), jnp.int32),
            pltpu.VMEM((_NBUF, _BT, D), jnp.float32),
            pltpu.VMEM((_NBUF, D // 8, 8, _TPAD), jnp.float32),
            pltpu.VMEM((S, D), jnp.float32),
        ] + [pltpu.SemaphoreType.DMA] * (3 * _NBUF),
    )


def kernel(x, token_table, pos_table):
    B, S = x.shape
    V, D = token_table.shape
    x4 = (x.astype(jnp.int32).T
          .reshape(S // 8, 8, B // _BT, _BT)
          .transpose(0, 2, 1, 3))
    out5 = _build(B, S, V, D)(x4, token_table, pos_table)
    return out5.transpose(2, 4, 0, 1, 3).reshape(B, S, D)


# speedup vs baseline: 4.7124x; 2.6177x over previous
import jax
import jax.numpy as jnp
from jax import lax
from jax.experimental import pallas as pl
from jax.experimental.pallas import tpu as pltpu
from jax.experimental.pallas import tpu_sc as plsc

_NC = 2
_NS = 16
_NW = _NC * _NS
_NBUF = 4
_BT = 128
_TPAD = 136


def _build(B, S, V, D):
    n_bt = B // _BT
    n_units = S * n_bt
    per_w = n_units // _NW
    mesh = plsc.VectorSubcoreMesh(core_axis_name="c", subcore_axis_name="s")

    def body(x_hbm, tab_hbm, pos_hbm, out_hbm, idx_v, rows_v, trans_v,
             pos_v, *sems):
        sg = sems[:_NBUF]
        ss = sems[_NBUF:2 * _NBUF]
        si = sems[2 * _NBUF:]
        wid = lax.axis_index("s") * _NC + lax.axis_index("c")
        base = wid * per_w

        pltpu.sync_copy(pos_hbm, pos_v)

        lane = lax.iota(jnp.int32, 16)
        idx_dt = [(c * 16 + lane) >> 3 for c in range(D // 16)]
        idx_dt = [(c * 16 + lane) >> 3 for c in range(D // 16)]
        idx_dr = [(c * 16 + lane) & 7 for c in range(D // 16)]

        def unit_sbt(i):
            uid = base + i
            s = uid // n_bt
            return s, uid % n_bt

        def load_idx(i, u):
            s, bt = unit_sbt(i)
            pltpu.async_copy(x_hbm.at[s >> 3, bt, s & 7], idx_v.at[u], si[u])

        def wait_idx(u):
            pltpu.make_async_copy(
                x_hbm.at[0, 0, 0], idx_v.at[u], si[u]).wait()

        def start_gather(u):
            pltpu.async_copy(tab_hbm.at[idx_v.at[u]], rows_v.at[u], sg[u])

        def wait_gather(u):
            pltpu.make_async_copy(
                tab_hbm.at[idx_v.at[u]], rows_v.at[u], sg[u]).wait()

        def start_scatter(i, u):
            s, bt = unit_sbt(i)
            pltpu.async_copy(trans_v.at[u, :, :, pl.ds(0, _BT)],
                             out_hbm.at[s, :, bt], ss[u])

        def wait_scatter(u):
            pltpu.make_async_copy(trans_v.at[u, :, :, pl.ds(0, _BT)],
                                  out_hbm.at[0, :, 0], ss[u]).wait()

        for u in range(_NBUF):
            load_idx(u, u)
        for u in range(2):
            wait_idx(u)
            start_gather(u)

        @pl.loop(0, per_w, step=_NBUF)
        def _(i0):
            for u in range(_NBUF):
                i = i0 + u
                un = (u + 2) % _NBUF

                @pl.when(i + 2 < per_w)
                def _():
                    wait_idx(un)
                    start_gather(un)

                @pl.when(i >= _NBUF)
                def _():
                    wait_scatter(u)

                wait_gather(u)

                @pl.when(i + _NBUF < per_w)
                def _():
                    load_idx(i + _NBUF, u)

                s, _bt = unit_sbt(i)
                pos_c = [pos_v[s, pl.ds(c * 16, 16)] for c in range(D // 16)]

                @plsc.parallel_loop(0, _BT, unroll=2)
                def _(t):
                    tv = jnp.full((16,), t, jnp.int32)
                    for c in range(D // 16):
                        v = rows_v[u, t, pl.ds(c * 16, 16)] + pos_c[c]
                        plsc.store_scatter(
                            trans_v.at[u], [idx_dt[c], idx_dr[c], tv], v)

                start_scatter(i, u)

        for u in range(_NBUF):
            wait_scatter((per_w - _NBUF + u) % _NBUF)

    return pl.kernel(
        body,
        out_type=jax.ShapeDtypeStruct((S, D // 8, B // _BT, 8, _BT),
                                      jnp.float32),
        mesh=mesh,
        compiler_params=pltpu.CompilerParams(
            use_tc_tiling_on_sc=False, needs_layout_passes=False),
        scratch_types=[
            pltpu.VMEM((_NBUF, _BT), jnp.int32),
            pltpu.VMEM((_NBUF, _BT, D), jnp.float32),
            pltpu.VMEM((_NBUF, D // 8, 8, _TPAD), jnp.float32),
            pltpu.VMEM((S, D), jnp.float32),
        ] + [pltpu.SemaphoreType.DMA] * (3 * _NBUF),
    )


def kernel(x, token_table, pos_table):
    B, S = x.shape
    V, D = token_table.shape
    x4 = (x.astype(jnp.int32).T
          .reshape(S // 8, 8, B // _BT, _BT)
          .transpose(0, 2, 1, 3))
    out5 = _build(B, S, V, D)(x4, token_table, pos_table)
    return out5.transpose(2, 4, 0, 1, 3).reshape(B, S, D)
